# baseline (speedup 1.0000x reference)
import functools

import jax
import jax.numpy as jnp
from jax import lax
from jax.experimental import pallas as pl
from jax.experimental.pallas import tpu as pltpu
from jax.experimental.pallas import tpu_sc as plsc

N = 4096
E = 131072
IN = 256
HID = 256
OUT = 64
K = 2048

BM = 512
BK = 1024
NI = N // BM
NK = N // BK

_STRIPE_ROWS = 256
_STRIPE_W = _STRIPE_ROWS * N
_EPT = E // 16
_TILE_W = _STRIPE_W // 16


def _build_c_kernel(src_hbm, dst_hbm, c_hbm, stripe_sp, zbuf, srcb, dstb,
                    idxb, valb, dmasem):
    core = lax.axis_index("c")
    tid = lax.axis_index("s")

    def zinit(i, carry):
        zbuf[pl.ds(i * 16, 16)] = jnp.zeros((16,), jnp.float32)
        return carry

    lax.fori_loop(0, 2048, zinit, 0)

    pltpu.sync_copy(src_hbm.at[pl.ds(tid * _EPT, _EPT)], srcb)
    pltpu.sync_copy(dst_hbm.at[pl.ds(tid * _EPT, _EPT)], dstb)

    def stripe_body(j, carry):
        sidx = core + 2 * j
        lo = sidx * _STRIPE_ROWS
        pltpu.sync_copy(zbuf, stripe_sp.at[pl.ds(tid * _TILE_W, 32768)])
        pltpu.sync_copy(zbuf, stripe_sp.at[pl.ds(tid * _TILE_W + 32768, 32768)])
        plsc.subcore_barrier()

        def prep(ch, c2):
            sv = srcb[pl.ds(ch * 16, 16)]
            dv = dstb[pl.ds(ch * 16, 16)]
            inb = (dv >= lo) & (dv < lo + _STRIPE_ROWS)
            dummy = ((tid * _EPT + ch * 16 + lax.iota(jnp.int32, 16)) * 61
                     ) & (_STRIPE_W - 1)
            idx = jnp.where(inb, (dv - lo) * N + sv, dummy)
            val = jnp.where(inb, jnp.ones((16,), jnp.float32),
                            jnp.zeros((16,), jnp.float32))
            idxb[pl.ds(ch * 16, 16)] = idx
            valb[pl.ds(ch * 16, 16)] = val
            return c2

        lax.fori_loop(0, _EPT // 16, prep, 0, unroll=8)
        pltpu.sync_copy(valb, stripe_sp.at[idxb], add=True)
        plsc.subcore_barrier()
        pltpu.sync_copy(
            stripe_sp.at[pl.ds(tid * _TILE_W, _TILE_W)],
            c_hbm.at[pl.ds(sidx * _STRIPE_W + tid * _TILE_W, _TILE_W)])
        plsc.subcore_barrier()
        return carry

    lax.fori_loop(0, 8, stripe_body, 0)


_build_c_cache = []


def _build_c(src, dst):
    if not _build_c_cache:
        _build_c_cache.append(functools.partial(
            pl.kernel,
            mesh=plsc.VectorSubcoreMesh(core_axis_name="c",
                                        subcore_axis_name="s"),
            out_type=jax.ShapeDtypeStruct((N * N,), jnp.float32),
            scratch_types=[
                pltpu.VMEM_SHARED((_STRIPE_W,), jnp.float32),
                pltpu.VMEM((32768,), jnp.float32),
                pltpu.VMEM((_EPT,), jnp.int32),
                pltpu.VMEM((_EPT,), jnp.int32),
                pltpu.VMEM((_EPT,), jnp.int32),
                pltpu.VMEM((_EPT,), jnp.float32),
                pltpu.SemaphoreType.DMA,
            ],
        )(_build_c_kernel))
    return _build_c_cache[0](src, dst)


def _eye(n):
    r = lax.broadcasted_iota(jnp.int32, (n, n), 0)
    c = lax.broadcasted_iota(jnp.int32, (n, n), 1)
    return (r == c).astype(jnp.float32)


def _prep_body(c_ref, dinv_ref, diag_ref):
    i, k = pl.program_id(0), pl.program_id(1)

    @pl.when(k == 0)
    def _():
        dinv_ref[...] = jnp.zeros_like(dinv_ref)
        diag_ref[...] = jnp.zeros_like(diag_ref)

    blk = c_ref[...]
    dinv_ref[...] += blk.sum(axis=1, keepdims=True)
    rg = lax.broadcasted_iota(jnp.int32, (BM, BK), 0) + i * BM
    cg = lax.broadcasted_iota(jnp.int32, (BM, BK), 1) + k * BK
    diag_ref[...] += (blk * (rg == cg).astype(jnp.float32)).sum(
        axis=1, keepdims=True)

    @pl.when(k == NK - 1)
    def _():
        dinv_ref[...] = lax.rsqrt(dinv_ref[...] + 2.0)


def _prep(C):
    return pl.pallas_call(
        _prep_body,
        grid=(NI, NK),
        in_specs=[pl.BlockSpec((BM, BK), lambda i, k: (i, k))],
        out_specs=(pl.BlockSpec((BM, 1), lambda i, k: (i, 0)),
                   pl.BlockSpec((BM, 1), lambda i, k: (i, 0))),
        out_shape=(jax.ShapeDtypeStruct((N, 1), jnp.float32),
                   jax.ShapeDtypeStruct((N, 1), jnp.float32)),
    )(C)


def _mm1_body(x_ref, w_ref, dinv_ref, xw_ref, h_ref):
    xw = jnp.dot(x_ref[...], w_ref[...], preferred_element_type=jnp.float32)
    xw_ref[...] = xw
    h_ref[...] = dinv_ref[...] * xw


def _mm1(x, W0, dinv):
    return pl.pallas_call(
        _mm1_body,
        grid=(NI,),
        in_specs=[pl.BlockSpec((BM, IN), lambda i: (i, 0)),
                  pl.BlockSpec((IN, HID), lambda i: (0, 0)),
                  pl.BlockSpec((BM, 1), lambda i: (i, 0))],
        out_specs=(pl.BlockSpec((BM, HID), lambda i: (i, 0)),
                   pl.BlockSpec((BM, HID), lambda i: (i, 0))),
        out_shape=(jax.ShapeDtypeStruct((N, HID), jnp.float32),
                   jax.ShapeDtypeStruct((N, HID), jnp.float32)),
    )(x, W0, dinv)


def _conv1_body(c_ref, h_ref, dinv_ref, xw_ref, b_ref, pw_ref,
                x0_ref, srow_ref):
    k = pl.program_id(1)

    @pl.when(k == 0)
    def _():
        x0_ref[...] = jnp.zeros_like(x0_ref)

    x0_ref[...] += jnp.dot(c_ref[...], h_ref[...],
                           preferred_element_type=jnp.float32)

    @pl.when(k == NK - 1)
    def _():
        dinv = dinv_ref[...]
        xw = xw_ref[...]
        pre = dinv * x0_ref[...] + (2.0 * dinv * dinv) * xw + b_ref[...]
        x0 = jnp.maximum(pre, 0.0)
        x0_ref[...] = x0
        pw = pw_ref[...]
        pwn = pw * lax.rsqrt(jnp.sum(pw * pw))
        s_col = jnp.tanh(jnp.dot(x0, pwn, preferred_element_type=jnp.float32))
        srow_ref[...] = (s_col * _eye(BM)).sum(axis=0, keepdims=True)


def _conv1(C, h, dinv, xw0, b0r, pw2):
    return pl.pallas_call(
        _conv1_body,
        grid=(NI, NK),
        in_specs=[pl.BlockSpec((BM, BK), lambda i, k: (i, k)),
                  pl.BlockSpec((BK, HID), lambda i, k: (k, 0)),
                  pl.BlockSpec((BM, 1), lambda i, k: (i, 0)),
                  pl.BlockSpec((BM, HID), lambda i, k: (i, 0)),
                  pl.BlockSpec((1, HID), lambda i, k: (0, 0)),
                  pl.BlockSpec((HID, 1), lambda i, k: (0, 0))],
        out_specs=(pl.BlockSpec((BM, HID), lambda i, k: (i, 0)),
                   pl.BlockSpec((1, BM), lambda i, k: (0, i))),
        out_shape=(jax.ShapeDtypeStruct((N, HID), jnp.float32),
                   jax.ShapeDtypeStruct((1, N), jnp.float32)),
    )(C, h, dinv, xw0, b0r, pw2)


def _pool_body(srow_ref, sblk_ref, x0_ref, w1_ref, mcol_ref, mrow_ref, y_ref):
    i = pl.program_id(0)
    s_all = srow_ref[...]
    s_blk = sblk_ref[...]
    s_col = (s_blk * _eye(BM)).sum(axis=1, keepdims=True)
    gt = (s_all > s_col).astype(jnp.float32).sum(axis=1, keepdims=True)
    jg = lax.broadcasted_iota(jnp.int32, (BM, N), 1)
    ig = lax.broadcasted_iota(jnp.int32, (BM, N), 0) + i * BM
    eqc = ((s_all == s_col) & (jg < ig)).astype(jnp.float32).sum(
        axis=1, keepdims=True)
    rank = gt + eqc
    m = (rank < float(K)).astype(jnp.float32)
    mcol_ref[...] = m
    mrow_ref[...] = (m * _eye(BM)).sum(axis=0, keepdims=True)
    y_ref[...] = jnp.dot(x0_ref[...] * (s_col * m), w1_ref[...],
                         preferred_element_type=jnp.float32)


def _pool(score_row, x0, W1):
    return pl.pallas_call(
        _pool_body,
        grid=(NI,),
        in_specs=[pl.BlockSpec((1, N), lambda i: (0, 0)),
                  pl.BlockSpec((1, BM), lambda i: (0, i)),
                  pl.BlockSpec((BM, HID), lambda i: (i, 0)),
                  pl.BlockSpec((HID, HID), lambda i: (0, 0))],
        out_specs=(pl.BlockSpec((BM, 1), lambda i: (i, 0)),
                   pl.BlockSpec((1, BM), lambda i: (0, i)),
                   pl.BlockSpec((BM, HID), lambda i: (i, 0))),
        out_shape=(jax.ShapeDtypeStruct((N, 1), jnp.float32),
                   jax.ShapeDtypeStruct((1, N), jnp.float32),
                   jax.ShapeDtypeStruct((N, HID), jnp.float32)),
    )(score_row, score_row, x0, W1)


def _stats_body(cik_ref, cki_ref, mrow_ref, mcol_ref, diag_ref,
                u1c_ref, u1r_ref, dfull_ref):
    i, k = pl.program_id(0), pl.program_id(1)

    @pl.when(k == 0)
    def _():
        u1c_ref[...] = jnp.zeros_like(u1c_ref)
        dfull_ref[...] = jnp.zeros_like(dfull_ref)

    cik = cik_ref[...]
    u1c_ref[...] += (cik * mrow_ref[...]).sum(axis=1, keepdims=True)
    prod = jnp.dot(cik, cki_ref[...], preferred_element_type=jnp.float32)
    dfull_ref[...] += (prod * _eye(BM)).sum(axis=1, keepdims=True)

    @pl.when(k == NK - 1)
    def _():
        c = diag_ref[...]
        u1 = u1c_ref[...] + (1.0 - c) * mcol_ref[...]
        u1c_ref[...] = u1
        u1r_ref[...] = (u1 * _eye(BM)).sum(axis=0, keepdims=True)
        dfull_ref[...] = 1.0 + dfull_ref[...] - c * c


def _stats(C, m_row, m_col, cdiag):
    return pl.pallas_call(
        _stats_body,
        grid=(NI, NK),
        in_specs=[pl.BlockSpec((BM, BK), lambda i, k: (i, k)),
                  pl.BlockSpec((BK, BM), lambda i, k: (k, i)),
                  pl.BlockSpec((1, BK), lambda i, k: (0, k)),
                  pl.BlockSpec((BM, 1), lambda i, k: (i, 0)),
                  pl.BlockSpec((BM, 1), lambda i, k: (i, 0))],
        out_specs=(pl.BlockSpec((BM, 1), lambda i, k: (i, 0)),
                   pl.BlockSpec((1, BM), lambda i, k: (0, i)),
                   pl.BlockSpec((BM, 1), lambda i, k: (i, 0))),
        out_shape=(jax.ShapeDtypeStruct((N, 1), jnp.float32),
                   jax.ShapeDtypeStruct((1, N), jnp.float32),
                   jax.ShapeDtypeStruct((N, 1), jnp.float32)),
    )(C, C, m_row, m_col, cdiag)


def _u2_body(c_ref, u1r_ref, u1c_ref, diag_ref, mcol_ref, dfull_ref, y_ref,
             dinvp_ref, z_ref):
    k = pl.program_id(1)

    @pl.when(k == 0)
    def _():
        dinvp_ref[...] = jnp.zeros_like(dinvp_ref)

    dinvp_ref[...] += (c_ref[...] * u1r_ref[...]).sum(axis=1, keepdims=True)

    @pl.when(k == NK - 1)
    def _():
        c = diag_ref[...]
        u2 = dinvp_ref[...] + (1.0 - c) * u1c_ref[...]
        deg_pool = mcol_ref[...] * (u2 - dfull_ref[...]) + 2.0
        dinvp = lax.rsqrt(deg_pool)
        dinvp_ref[...] = dinvp
        z_ref[...] = dinvp * y_ref[...]


def _u2(C, u1r, u1c, cdiag, m_col, dfull, y):
    return pl.pallas_call(
        _u2_body,
        grid=(NI, NK),
        in_specs=[pl.BlockSpec((BM, BK), lambda i, k: (i, k)),
                  pl.BlockSpec((1, BK), lambda i, k: (0, k)),
                  pl.BlockSpec((BM, 1), lambda i, k: (i, 0)),
                  pl.BlockSpec((BM, 1), lambda i, k: (i, 0)),
                  pl.BlockSpec((BM, 1), lambda i, k: (i, 0)),
                  pl.BlockSpec((BM, 1), lambda i, k: (i, 0)),
                  pl.BlockSpec((BM, HID), lambda i, k: (i, 0))],
        out_specs=(pl.BlockSpec((BM, 1), lambda i, k: (i, 0)),
                   pl.BlockSpec((BM, HID), lambda i, k: (i, 0))),
        out_shape=(jax.ShapeDtypeStruct((N, 1), jnp.float32),
                   jax.ShapeDtypeStruct((N, HID), jnp.float32)),
    )(C, u1r, u1c, cdiag, m_col, dfull, y)


def _t1_body(c_ref, zk_ref, zi_ref, diag_ref, t1_ref):
    k = pl.program_id(1)

    @pl.when(k == 0)
    def _():
        t1_ref[...] = jnp.zeros_like(t1_ref)

    t1_ref[...] += jnp.dot(c_ref[...], zk_ref[...],
                           preferred_element_type=jnp.float32)

    @pl.when(k == NK - 1)
    def _():
        t1_ref[...] += (1.0 - diag_ref[...]) * zi_ref[...]


def _t1(C, z, cdiag):
    return pl.pallas_call(
        _t1_body,
        grid=(NI, NK),
        in_specs=[pl.BlockSpec((BM, BK), lambda i, k: (i, k)),
                  pl.BlockSpec((BK, HID), lambda i, k: (k, 0)),
                  pl.BlockSpec((BM, HID), lambda i, k: (i, 0)),
                  pl.BlockSpec((BM, 1), lambda i, k: (i, 0))],
        out_specs=pl.BlockSpec((BM, HID), lambda i, k: (i, 0)),
        out_shape=jax.ShapeDtypeStruct((N, HID), jnp.float32),
    )(C, z, z, cdiag)


def _t2_body(c_ref, t1k_ref, t1i_ref, diag_ref, dinvp_ref, dfull_ref,
             zi_ref, mcol_ref, x0_ref, b1_ref, xr_ref):
    k = pl.program_id(1)

    @pl.when(k == 0)
    def _():
        xr_ref[...] = jnp.zeros_like(xr_ref)

    xr_ref[...] += jnp.dot(c_ref[...], t1k_ref[...],
                           preferred_element_type=jnp.float32)

    @pl.when(k == NK - 1)
    def _():
        z = zi_ref[...]
        t2 = xr_ref[...] + (1.0 - diag_ref[...]) * t1i_ref[...]
        outp = dinvp_ref[...] * (t2 - dfull_ref[...] * z + 2.0 * z) + b1_ref[...]
        xr_ref[...] = x0_ref[...] + mcol_ref[...] * jnp.maximum(outp, 0.0)


def _t2(C, t1, cdiag, dinvp, dfull, z, m_col, x0, b1r):
    return pl.pallas_call(
        _t2_body,
        grid=(NI, NK),
        in_specs=[pl.BlockSpec((BM, BK), lambda i, k: (i, k)),
                  pl.BlockSpec((BK, HID), lambda i, k: (k, 0)),
                  pl.BlockSpec((BM, HID), lambda i, k: (i, 0)),
                  pl.BlockSpec((BM, 1), lambda i, k: (i, 0)),
                  pl.BlockSpec((BM, 1), lambda i, k: (i, 0)),
                  pl.BlockSpec((BM, 1), lambda i, k: (i, 0)),
                  pl.BlockSpec((BM, HID), lambda i, k: (i, 0)),
                  pl.BlockSpec((BM, 1), lambda i, k: (i, 0)),
                  pl.BlockSpec((BM, HID), lambda i, k: (i, 0)),
                  pl.BlockSpec((1, HID), lambda i, k: (0, 0))],
        out_specs=pl.BlockSpec((BM, HID), lambda i, k: (i, 0)),
        out_shape=jax.ShapeDtypeStruct((N, HID), jnp.float32),
    )(C, t1, t1, cdiag, dinvp, dfull, z, m_col, x0, b1r)


def _mm2_body(xr_ref, w2_ref, dinv_ref, xw2_ref, g_ref):
    xw2 = jnp.dot(xr_ref[...], w2_ref[...], preferred_element_type=jnp.float32)
    xw2_ref[...] = xw2
    g_ref[...] = dinv_ref[...] * xw2


def _mm2(xr, W2, dinv):
    return pl.pallas_call(
        _mm2_body,
        grid=(NI,),
        in_specs=[pl.BlockSpec((BM, HID), lambda i: (i, 0)),
                  pl.BlockSpec((HID, OUT), lambda i: (0, 0)),
                  pl.BlockSpec((BM, 1), lambda i: (i, 0))],
        out_specs=(pl.BlockSpec((BM, OUT), lambda i: (i, 0)),
                   pl.BlockSpec((BM, OUT), lambda i: (i, 0))),
        out_shape=(jax.ShapeDtypeStruct((N, OUT), jnp.float32),
                   jax.ShapeDtypeStruct((N, OUT), jnp.float32)),
    )(xr, W2, dinv)


def _conv3_body(c_ref, g_ref, dinv_ref, xw2_ref, b_ref, out_ref):
    k = pl.program_id(1)

    @pl.when(k == 0)
    def _():
        out_ref[...] = jnp.zeros_like(out_ref)

    out_ref[...] += jnp.dot(c_ref[...], g_ref[...],
                            preferred_element_type=jnp.float32)

    @pl.when(k == NK - 1)
    def _():
        dinv = dinv_ref[...]
        o = (dinv * out_ref[...] + (2.0 * dinv * dinv) * xw2_ref[...]
             + b_ref[...])
        mx = jnp.max(o, axis=1, keepdims=True)
        out_ref[...] = o - mx - jnp.log(
            jnp.sum(jnp.exp(o - mx), axis=1, keepdims=True))


def _conv3(C, g, dinv, xw2, b2r):
    return pl.pallas_call(
        _conv3_body,
        grid=(NI, NK),
        in_specs=[pl.BlockSpec((BM, BK), lambda i, k: (i, k)),
                  pl.BlockSpec((BK, OUT), lambda i, k: (k, 0)),
                  pl.BlockSpec((BM, 1), lambda i, k: (i, 0)),
                  pl.BlockSpec((BM, OUT), lambda i, k: (i, 0)),
                  pl.BlockSpec((1, OUT), lambda i, k: (0, 0))],
        out_specs=pl.BlockSpec((BM, OUT), lambda i, k: (i, 0)),
        out_shape=jax.ShapeDtypeStruct((N, OUT), jnp.float32),
    )(C, g, dinv, xw2, b2r)


def kernel(x, edge_index, W0, b0, pw, W1, b1, W2, b2):
    src = edge_index[0]
    dst = edge_index[1]
    cflat = _build_c(src, dst)
    C = cflat.reshape(N, N)

    dinv, cdiag = _prep(C)
    xw0, h = _mm1(x, W0, dinv)
    x0, score_row = _conv1(C, h, dinv, xw0, b0.reshape(1, HID),
                           pw.reshape(HID, 1))
    m_col, m_row, y = _pool(score_row, x0, W1)
    u1c, u1r, dfull = _stats(C, m_row, m_col, cdiag)
    dinvp, z = _u2(C, u1r, u1c, cdiag, m_col, dfull, y)
    t1 = _t1(C, z, cdiag)
    xr = _t2(C, t1, cdiag, dinvp, dfull, z, m_col, x0, b1.reshape(1, HID))
    xw2, g = _mm2(xr, W2, dinv)
    return _conv3(C, g, dinv, xw2, b2.reshape(1, OUT))


# speedup vs baseline: 1.0774x; 1.0774x over previous
import functools

import jax
import jax.numpy as jnp
from jax import lax
from jax.experimental import pallas as pl
from jax.experimental.pallas import tpu as pltpu
from jax.experimental.pallas import tpu_sc as plsc

N = 4096
E = 131072
IN = 256
HID = 256
OUT = 64
K = 2048

BM = 512
BK = 1024
NI = N // BM
NK = N // BK

_STRIPE_ROWS = 256
_STRIPE_W = _STRIPE_ROWS * N
_EPT = E // 16
_TILE_W = _STRIPE_W // 16


def _build_c_kernel(src_hbm, dst_hbm, c_hbm, stripe_sp, zbuf, srcb, dstb,
                    idxb, valb, dmasem):
    core = lax.axis_index("c")
    tid = lax.axis_index("s")

    def zinit(i, carry):
        zbuf[pl.ds(i * 16, 16)] = jnp.zeros((16,), jnp.float32)
        return carry

    lax.fori_loop(0, 2048, zinit, 0)

    pltpu.sync_copy(src_hbm.at[pl.ds(tid * _EPT, _EPT)], srcb)
    pltpu.sync_copy(dst_hbm.at[pl.ds(tid * _EPT, _EPT)], dstb)

    def stripe_body(j, carry):
        sidx = core + 2 * j
        lo = sidx * _STRIPE_ROWS
        pltpu.sync_copy(zbuf, stripe_sp.at[pl.ds(tid * _TILE_W, 32768)])
        pltpu.sync_copy(zbuf, stripe_sp.at[pl.ds(tid * _TILE_W + 32768, 32768)])
        plsc.subcore_barrier()

        def prep(ch, c2):
            sv = srcb[pl.ds(ch * 16, 16)]
            dv = dstb[pl.ds(ch * 16, 16)]
            inb = (dv >= lo) & (dv < lo + _STRIPE_ROWS)
            dummy = ((tid * _EPT + ch * 16 + lax.iota(jnp.int32, 16)) * 61
                     ) & (_STRIPE_W - 1)
            idx = jnp.where(inb, (dv - lo) * N + sv, dummy)
            val = jnp.where(inb, jnp.ones((16,), jnp.float32),
                            jnp.zeros((16,), jnp.float32))
            idxb[pl.ds(ch * 16, 16)] = idx
            valb[pl.ds(ch * 16, 16)] = val
            return c2

        lax.fori_loop(0, _EPT // 16, prep, 0)
        pltpu.sync_copy(valb, stripe_sp.at[idxb], add=True)
        plsc.subcore_barrier()
        pltpu.sync_copy(
            stripe_sp.at[pl.ds(tid * _TILE_W, _TILE_W)],
            c_hbm.at[pl.ds(sidx * _STRIPE_W + tid * _TILE_W, _TILE_W)])
        plsc.subcore_barrier()
        return carry

    lax.fori_loop(0, 8, stripe_body, 0)


_build_c_cache = []


def _build_c(src, dst):
    if not _build_c_cache:
        _build_c_cache.append(functools.partial(
            pl.kernel,
            mesh=plsc.VectorSubcoreMesh(core_axis_name="c",
                                        subcore_axis_name="s"),
            out_type=jax.ShapeDtypeStruct((N * N,), jnp.float32),
            scratch_types=[
                pltpu.VMEM_SHARED((_STRIPE_W,), jnp.float32),
                pltpu.VMEM((32768,), jnp.float32),
                pltpu.VMEM((_EPT,), jnp.int32),
                pltpu.VMEM((_EPT,), jnp.int32),
                pltpu.VMEM((_EPT,), jnp.int32),
                pltpu.VMEM((_EPT,), jnp.float32),
                pltpu.SemaphoreType.DMA,
            ],
        )(_build_c_kernel))
    return _build_c_cache[0](src, dst)


def _eye(n):
    r = lax.broadcasted_iota(jnp.int32, (n, n), 0)
    c = lax.broadcasted_iota(jnp.int32, (n, n), 1)
    return (r == c).astype(jnp.float32)


def _prep_body(c_ref, dinv_ref, diag_ref, cbf_ref):
    i, k = pl.program_id(0), pl.program_id(1)

    @pl.when(k == 0)
    def _():
        dinv_ref[...] = jnp.zeros_like(dinv_ref)
        diag_ref[...] = jnp.zeros_like(diag_ref)

    blk = c_ref[...]
    cbf_ref[...] = blk.astype(jnp.bfloat16)
    dinv_ref[...] += blk.sum(axis=1, keepdims=True)
    rg = lax.broadcasted_iota(jnp.int32, (BM, BK), 0) + i * BM
    cg = lax.broadcasted_iota(jnp.int32, (BM, BK), 1) + k * BK
    diag_ref[...] += (blk * (rg == cg).astype(jnp.float32)).sum(
        axis=1, keepdims=True)

    @pl.when(k == NK - 1)
    def _():
        dinv_ref[...] = lax.rsqrt(dinv_ref[...] + 2.0)


def _prep(C):
    return pl.pallas_call(
        _prep_body,
        grid=(NI, NK),
        in_specs=[pl.BlockSpec((BM, BK), lambda i, k: (i, k))],
        out_specs=(pl.BlockSpec((BM, 1), lambda i, k: (i, 0)),
                   pl.BlockSpec((BM, 1), lambda i, k: (i, 0)),
                   pl.BlockSpec((BM, BK), lambda i, k: (i, k))),
        out_shape=(jax.ShapeDtypeStruct((N, 1), jnp.float32),
                   jax.ShapeDtypeStruct((N, 1), jnp.float32),
                   jax.ShapeDtypeStruct((N, N), jnp.bfloat16)),
    )(C)


def _mm0_body(x_ref, w_ref, xw_ref):
    xw_ref[...] = jnp.dot(x_ref[...], w_ref[...],
                          preferred_element_type=jnp.float32)


def _mm0(x, W0):
    return pl.pallas_call(
        _mm0_body,
        grid=(NI,),
        in_specs=[pl.BlockSpec((BM, IN), lambda i: (i, 0)),
                  pl.BlockSpec((IN, HID), lambda i: (0, 0))],
        out_specs=pl.BlockSpec((BM, HID), lambda i: (i, 0)),
        out_shape=jax.ShapeDtypeStruct((N, HID), jnp.float32),
    )(x, W0)


def _conv1_body(c_ref, xwk_ref, dinvk_ref, dinv_ref, xw_ref, b_ref, pw_ref,
                x0_ref, srow_ref):
    k = pl.program_id(1)

    @pl.when(k == 0)
    def _():
        x0_ref[...] = jnp.zeros_like(x0_ref)

    x0_ref[...] += jnp.dot(c_ref[...].astype(jnp.float32),
                           dinvk_ref[...] * xwk_ref[...],
                           preferred_element_type=jnp.float32)

    @pl.when(k == NK - 1)
    def _():
        dinv = dinv_ref[...]
        xw = xw_ref[...]
        pre = dinv * x0_ref[...] + (2.0 * dinv * dinv) * xw + b_ref[...]
        x0 = jnp.maximum(pre, 0.0)
        x0_ref[...] = x0
        pw = pw_ref[...]
        pwn = pw * lax.rsqrt(jnp.sum(pw * pw))
        s_col = jnp.tanh(jnp.dot(x0, pwn, preferred_element_type=jnp.float32))
        srow_ref[...] = (s_col * _eye(BM)).sum(axis=0, keepdims=True)


def _conv1(C, dinv, xw0, b0r, pw2):
    return pl.pallas_call(
        _conv1_body,
        grid=(NI, NK),
        in_specs=[pl.BlockSpec((BM, BK), lambda i, k: (i, k)),
                  pl.BlockSpec((BK, HID), lambda i, k: (k, 0)),
                  pl.BlockSpec((BK, 1), lambda i, k: (k, 0)),
                  pl.BlockSpec((BM, 1), lambda i, k: (i, 0)),
                  pl.BlockSpec((BM, HID), lambda i, k: (i, 0)),
                  pl.BlockSpec((1, HID), lambda i, k: (0, 0)),
                  pl.BlockSpec((HID, 1), lambda i, k: (0, 0))],
        out_specs=(pl.BlockSpec((BM, HID), lambda i, k: (i, 0)),
                   pl.BlockSpec((1, BM), lambda i, k: (0, i))),
        out_shape=(jax.ShapeDtypeStruct((N, HID), jnp.float32),
                   jax.ShapeDtypeStruct((1, N), jnp.float32)),
    )(C, xw0, dinv, dinv, xw0, b0r, pw2)


def _pool_body(srow_ref, sblk_ref, x0_ref, w1_ref, mcol_ref, mrow_ref, y_ref):
    i = pl.program_id(0)
    s_all = srow_ref[...]
    s_blk = sblk_ref[...]
    s_col = (s_blk * _eye(BM)).sum(axis=1, keepdims=True)
    gt = (s_all > s_col).astype(jnp.float32).sum(axis=1, keepdims=True)
    jg = lax.broadcasted_iota(jnp.int32, (BM, N), 1)
    ig = lax.broadcasted_iota(jnp.int32, (BM, N), 0) + i * BM
    eqc = ((s_all == s_col) & (jg < ig)).astype(jnp.float32).sum(
        axis=1, keepdims=True)
    rank = gt + eqc
    m = (rank < float(K)).astype(jnp.float32)
    mcol_ref[...] = m
    mrow_ref[...] = (m * _eye(BM)).sum(axis=0, keepdims=True)
    y_ref[...] = jnp.dot(x0_ref[...] * (s_col * m), w1_ref[...],
                         preferred_element_type=jnp.float32)


def _pool(score_row, x0, W1):
    return pl.pallas_call(
        _pool_body,
        grid=(NI,),
        in_specs=[pl.BlockSpec((1, N), lambda i: (0, 0)),
                  pl.BlockSpec((1, BM), lambda i: (0, i)),
                  pl.BlockSpec((BM, HID), lambda i: (i, 0)),
                  pl.BlockSpec((HID, HID), lambda i: (0, 0))],
        out_specs=(pl.BlockSpec((BM, 1), lambda i: (i, 0)),
                   pl.BlockSpec((1, BM), lambda i: (0, i)),
                   pl.BlockSpec((BM, HID), lambda i: (i, 0))),
        out_shape=(jax.ShapeDtypeStruct((N, 1), jnp.float32),
                   jax.ShapeDtypeStruct((1, N), jnp.float32),
                   jax.ShapeDtypeStruct((N, HID), jnp.float32)),
    )(score_row, score_row, x0, W1)


def _stats_body(cik_ref, cki_ref, mrow_ref, mcol_ref, diag_ref,
                u1c_ref, u1r_ref, dfull_ref):
    i, k = pl.program_id(0), pl.program_id(1)

    @pl.when(k == 0)
    def _():
        u1c_ref[...] = jnp.zeros_like(u1c_ref)
        dfull_ref[...] = jnp.zeros_like(dfull_ref)

    cik = cik_ref[...].astype(jnp.float32)
    u1c_ref[...] += (cik * mrow_ref[...]).sum(axis=1, keepdims=True)
    prod = jnp.dot(cik, cki_ref[...].astype(jnp.float32),
                   preferred_element_type=jnp.float32)
    dfull_ref[...] += (prod * _eye(BM)).sum(axis=1, keepdims=True)

    @pl.when(k == NK - 1)
    def _():
        c = diag_ref[...]
        u1 = u1c_ref[...] + (1.0 - c) * mcol_ref[...]
        u1c_ref[...] = u1
        u1r_ref[...] = (u1 * _eye(BM)).sum(axis=0, keepdims=True)
        dfull_ref[...] = 1.0 + dfull_ref[...] - c * c


def _stats(C, m_row, m_col, cdiag):
    return pl.pallas_call(
        _stats_body,
        grid=(NI, NK),
        in_specs=[pl.BlockSpec((BM, BK), lambda i, k: (i, k)),
                  pl.BlockSpec((BK, BM), lambda i, k: (k, i)),
                  pl.BlockSpec((1, BK), lambda i, k: (0, k)),
                  pl.BlockSpec((BM, 1), lambda i, k: (i, 0)),
                  pl.BlockSpec((BM, 1), lambda i, k: (i, 0))],
        out_specs=(pl.BlockSpec((BM, 1), lambda i, k: (i, 0)),
                   pl.BlockSpec((1, BM), lambda i, k: (0, i)),
                   pl.BlockSpec((BM, 1), lambda i, k: (i, 0))),
        out_shape=(jax.ShapeDtypeStruct((N, 1), jnp.float32),
                   jax.ShapeDtypeStruct((1, N), jnp.float32),
                   jax.ShapeDtypeStruct((N, 1), jnp.float32)),
    )(C, C, m_row, m_col, cdiag)


def _u2_body(c_ref, u1r_ref, u1c_ref, diag_ref, mcol_ref, dfull_ref, y_ref,
             dinvp_ref, z_ref):
    k = pl.program_id(1)

    @pl.when(k == 0)
    def _():
        dinvp_ref[...] = jnp.zeros_like(dinvp_ref)

    dinvp_ref[...] += (c_ref[...].astype(jnp.float32)
                       * u1r_ref[...]).sum(axis=1, keepdims=True)

    @pl.when(k == NK - 1)
    def _():
        c = diag_ref[...]
        u2 = dinvp_ref[...] + (1.0 - c) * u1c_ref[...]
        deg_pool = mcol_ref[...] * (u2 - dfull_ref[...]) + 2.0
        dinvp = lax.rsqrt(deg_pool)
        dinvp_ref[...] = dinvp
        z_ref[...] = dinvp * y_ref[...]


def _u2(C, u1r, u1c, cdiag, m_col, dfull, y):
    return pl.pallas_call(
        _u2_body,
        grid=(NI, NK),
        in_specs=[pl.BlockSpec((BM, BK), lambda i, k: (i, k)),
                  pl.BlockSpec((1, BK), lambda i, k: (0, k)),
                  pl.BlockSpec((BM, 1), lambda i, k: (i, 0)),
                  pl.BlockSpec((BM, 1), lambda i, k: (i, 0)),
                  pl.BlockSpec((BM, 1), lambda i, k: (i, 0)),
                  pl.BlockSpec((BM, 1), lambda i, k: (i, 0)),
                  pl.BlockSpec((BM, HID), lambda i, k: (i, 0))],
        out_specs=(pl.BlockSpec((BM, 1), lambda i, k: (i, 0)),
                   pl.BlockSpec((BM, HID), lambda i, k: (i, 0))),
        out_shape=(jax.ShapeDtypeStruct((N, 1), jnp.float32),
                   jax.ShapeDtypeStruct((N, HID), jnp.float32)),
    )(C, u1r, u1c, cdiag, m_col, dfull, y)


def _t1_body(c_ref, zk_ref, zi_ref, diag_ref, t1_ref):
    k = pl.program_id(1)

    @pl.when(k == 0)
    def _():
        t1_ref[...] = jnp.zeros_like(t1_ref)

    t1_ref[...] += jnp.dot(c_ref[...].astype(jnp.float32), zk_ref[...],
                           preferred_element_type=jnp.float32)

    @pl.when(k == NK - 1)
    def _():
        t1_ref[...] += (1.0 - diag_ref[...]) * zi_ref[...]


def _t1(C, z, cdiag):
    return pl.pallas_call(
        _t1_body,
        grid=(NI, NK),
        in_specs=[pl.BlockSpec((BM, BK), lambda i, k: (i, k)),
                  pl.BlockSpec((BK, HID), lambda i, k: (k, 0)),
                  pl.BlockSpec((BM, HID), lambda i, k: (i, 0)),
                  pl.BlockSpec((BM, 1), lambda i, k: (i, 0))],
        out_specs=pl.BlockSpec((BM, HID), lambda i, k: (i, 0)),
        out_shape=jax.ShapeDtypeStruct((N, HID), jnp.float32),
    )(C, z, z, cdiag)


def _t2_body(c_ref, t1k_ref, t1i_ref, diag_ref, dinvp_ref, dfull_ref,
             zi_ref, mcol_ref, x0_ref, b1_ref, xr_ref):
    k = pl.program_id(1)

    @pl.when(k == 0)
    def _():
        xr_ref[...] = jnp.zeros_like(xr_ref)

    xr_ref[...] += jnp.dot(c_ref[...].astype(jnp.float32), t1k_ref[...],
                           preferred_element_type=jnp.float32)

    @pl.when(k == NK - 1)
    def _():
        z = zi_ref[...]
        t2 = xr_ref[...] + (1.0 - diag_ref[...]) * t1i_ref[...]
        outp = dinvp_ref[...] * (t2 - dfull_ref[...] * z + 2.0 * z) + b1_ref[...]
        xr_ref[...] = x0_ref[...] + mcol_ref[...] * jnp.maximum(outp, 0.0)


def _t2(C, t1, cdiag, dinvp, dfull, z, m_col, x0, b1r):
    return pl.pallas_call(
        _t2_body,
        grid=(NI, NK),
        in_specs=[pl.BlockSpec((BM, BK), lambda i, k: (i, k)),
                  pl.BlockSpec((BK, HID), lambda i, k: (k, 0)),
                  pl.BlockSpec((BM, HID), lambda i, k: (i, 0)),
                  pl.BlockSpec((BM, 1), lambda i, k: (i, 0)),
                  pl.BlockSpec((BM, 1), lambda i, k: (i, 0)),
                  pl.BlockSpec((BM, 1), lambda i, k: (i, 0)),
                  pl.BlockSpec((BM, HID), lambda i, k: (i, 0)),
                  pl.BlockSpec((BM, 1), lambda i, k: (i, 0)),
                  pl.BlockSpec((BM, HID), lambda i, k: (i, 0)),
                  pl.BlockSpec((1, HID), lambda i, k: (0, 0))],
        out_specs=pl.BlockSpec((BM, HID), lambda i, k: (i, 0)),
        out_shape=jax.ShapeDtypeStruct((N, HID), jnp.float32),
    )(C, t1, t1, cdiag, dinvp, dfull, z, m_col, x0, b1r)


def _mm2_body(xr_ref, w2_ref, dinv_ref, xw2_ref, g_ref):
    xw2 = jnp.dot(xr_ref[...], w2_ref[...], preferred_element_type=jnp.float32)
    xw2_ref[...] = xw2
    g_ref[...] = dinv_ref[...] * xw2


def _mm2(xr, W2, dinv):
    return pl.pallas_call(
        _mm2_body,
        grid=(NI,),
        in_specs=[pl.BlockSpec((BM, HID), lambda i: (i, 0)),
                  pl.BlockSpec((HID, OUT), lambda i: (0, 0)),
                  pl.BlockSpec((BM, 1), lambda i: (i, 0))],
        out_specs=(pl.BlockSpec((BM, OUT), lambda i: (i, 0)),
                   pl.BlockSpec((BM, OUT), lambda i: (i, 0))),
        out_shape=(jax.ShapeDtypeStruct((N, OUT), jnp.float32),
                   jax.ShapeDtypeStruct((N, OUT), jnp.float32)),
    )(xr, W2, dinv)


def _conv3_body(c_ref, g_ref, dinv_ref, xw2_ref, b_ref, out_ref):
    k = pl.program_id(1)

    @pl.when(k == 0)
    def _():
        out_ref[...] = jnp.zeros_like(out_ref)

    out_ref[...] += jnp.dot(c_ref[...].astype(jnp.float32), g_ref[...],
                            preferred_element_type=jnp.float32)

    @pl.when(k == NK - 1)
    def _():
        dinv = dinv_ref[...]
        o = (dinv * out_ref[...] + (2.0 * dinv * dinv) * xw2_ref[...]
             + b_ref[...])
        mx = jnp.max(o, axis=1, keepdims=True)
        out_ref[...] = o - mx - jnp.log(
            jnp.sum(jnp.exp(o - mx), axis=1, keepdims=True))


def _conv3(C, g, dinv, xw2, b2r):
    return pl.pallas_call(
        _conv3_body,
        grid=(NI, NK),
        in_specs=[pl.BlockSpec((BM, BK), lambda i, k: (i, k)),
                  pl.BlockSpec((BK, OUT), lambda i, k: (k, 0)),
                  pl.BlockSpec((BM, 1), lambda i, k: (i, 0)),
                  pl.BlockSpec((BM, OUT), lambda i, k: (i, 0)),
                  pl.BlockSpec((1, OUT), lambda i, k: (0, 0))],
        out_specs=pl.BlockSpec((BM, OUT), lambda i, k: (i, 0)),
        out_shape=jax.ShapeDtypeStruct((N, OUT), jnp.float32),
    )(C, g, dinv, xw2, b2r)


def kernel(x, edge_index, W0, b0, pw, W1, b1, W2, b2):
    src = edge_index[0]
    dst = edge_index[1]
    cflat = _build_c(src, dst)
    C = cflat.reshape(N, N)

    xw0 = _mm0(x, W0)
    dinv, cdiag, Cbf = _prep(C)
    x0, score_row = _conv1(Cbf, dinv, xw0, b0.reshape(1, HID),
                           pw.reshape(HID, 1))
    m_col, m_row, y = _pool(score_row, x0, W1)
    u1c, u1r, dfull = _stats(Cbf, m_row, m_col, cdiag)
    dinvp, z = _u2(Cbf, u1r, u1c, cdiag, m_col, dfull, y)
    t1 = _t1(Cbf, z, cdiag)
    xr = _t2(Cbf, t1, cdiag, dinvp, dfull, z, m_col, x0, b1.reshape(1, HID))
    xw2, g = _mm2(xr, W2, dinv)
    return _conv3(Cbf, g, dinv, xw2, b2.reshape(1, OUT))


# speedup vs baseline: 1.2269x; 1.1387x over previous
import functools

import jax
import jax.numpy as jnp
from jax import lax
from jax.experimental import pallas as pl
from jax.experimental.pallas import tpu as pltpu
from jax.experimental.pallas import tpu_sc as plsc

N = 4096
E = 131072
IN = 256
HID = 256
OUT = 64
K = 2048

BM = 512
BK = 2048
NI = N // BM
NK = N // BK

_STRIPE_ROWS = 256
_STRIPE_W = _STRIPE_ROWS * N
_EPT = E // 16
_TILE_W = _STRIPE_W // 16


def _build_c_kernel(src_hbm, dst_hbm, c_hbm, stripe_sp, zbuf, srcb, dstb,
                    idxb, valb, dmasem):
    core = lax.axis_index("c")
    tid = lax.axis_index("s")

    def zinit(i, carry):
        zbuf[pl.ds(i * 16, 16)] = jnp.zeros((16,), jnp.float32)
        return carry

    lax.fori_loop(0, 2048, zinit, 0)

    pltpu.sync_copy(src_hbm.at[pl.ds(tid * _EPT, _EPT)], srcb)
    pltpu.sync_copy(dst_hbm.at[pl.ds(tid * _EPT, _EPT)], dstb)

    def stripe_body(j, carry):
        sidx = core + 2 * j
        lo = sidx * _STRIPE_ROWS
        pltpu.sync_copy(zbuf, stripe_sp.at[pl.ds(tid * _TILE_W, 32768)])
        pltpu.sync_copy(zbuf, stripe_sp.at[pl.ds(tid * _TILE_W + 32768, 32768)])
        plsc.subcore_barrier()

        def prep(ch, c2):
            sv = srcb[pl.ds(ch * 16, 16)]
            dv = dstb[pl.ds(ch * 16, 16)]
            inb = (dv >= lo) & (dv < lo + _STRIPE_ROWS)
            dummy = ((tid * _EPT + ch * 16 + lax.iota(jnp.int32, 16)) * 61
                     ) & (_STRIPE_W - 1)
            idx = jnp.where(inb, (dv - lo) * N + sv, dummy)
            val = jnp.where(inb, jnp.ones((16,), jnp.float32),
                            jnp.zeros((16,), jnp.float32))
            idxb[pl.ds(ch * 16, 16)] = idx
            valb[pl.ds(ch * 16, 16)] = val
            return c2

        lax.fori_loop(0, _EPT // 16, prep, 0)
        pltpu.sync_copy(valb, stripe_sp.at[idxb], add=True)
        plsc.subcore_barrier()
        pltpu.sync_copy(
            stripe_sp.at[pl.ds(tid * _TILE_W, _TILE_W)],
            c_hbm.at[pl.ds(sidx * _STRIPE_W + tid * _TILE_W, _TILE_W)])
        plsc.subcore_barrier()
        return carry

    lax.fori_loop(0, 8, stripe_body, 0)


_build_c_cache = []


def _build_c(src, dst):
    if not _build_c_cache:
        _build_c_cache.append(functools.partial(
            pl.kernel,
            mesh=plsc.VectorSubcoreMesh(core_axis_name="c",
                                        subcore_axis_name="s"),
            out_type=jax.ShapeDtypeStruct((N * N,), jnp.float32),
            scratch_types=[
                pltpu.VMEM_SHARED((_STRIPE_W,), jnp.float32),
                pltpu.VMEM((32768,), jnp.float32),
                pltpu.VMEM((_EPT,), jnp.int32),
                pltpu.VMEM((_EPT,), jnp.int32),
                pltpu.VMEM((_EPT,), jnp.int32),
                pltpu.VMEM((_EPT,), jnp.float32),
                pltpu.SemaphoreType.DMA,
            ],
        )(_build_c_kernel))
    return _build_c_cache[0](src, dst)


def _eye(n):
    r = lax.broadcasted_iota(jnp.int32, (n, n), 0)
    c = lax.broadcasted_iota(jnp.int32, (n, n), 1)
    return (r == c).astype(jnp.float32)


def _prep_body(c_ref, dinv_ref, diag_ref, cbf_ref):
    i, k = pl.program_id(0), pl.program_id(1)

    @pl.when(k == 0)
    def _():
        dinv_ref[...] = jnp.zeros_like(dinv_ref)
        diag_ref[...] = jnp.zeros_like(diag_ref)

    blk = c_ref[...]
    cbf_ref[...] = blk.astype(jnp.bfloat16)
    dinv_ref[...] += blk.sum(axis=1, keepdims=True)
    rg = lax.broadcasted_iota(jnp.int32, (BM, BK), 0) + i * BM
    cg = lax.broadcasted_iota(jnp.int32, (BM, BK), 1) + k * BK
    diag_ref[...] += (blk * (rg == cg).astype(jnp.float32)).sum(
        axis=1, keepdims=True)

    @pl.when(k == NK - 1)
    def _():
        dinv_ref[...] = lax.rsqrt(dinv_ref[...] + 2.0)


def _prep(C):
    return pl.pallas_call(
        _prep_body,
        grid=(NI, NK),
        in_specs=[pl.BlockSpec((BM, BK), lambda i, k: (i, k))],
        out_specs=(pl.BlockSpec((BM, 1), lambda i, k: (i, 0)),
                   pl.BlockSpec((BM, 1), lambda i, k: (i, 0)),
                   pl.BlockSpec((BM, BK), lambda i, k: (i, k))),
        out_shape=(jax.ShapeDtypeStruct((N, 1), jnp.float32),
                   jax.ShapeDtypeStruct((N, 1), jnp.float32),
                   jax.ShapeDtypeStruct((N, N), jnp.bfloat16)),
    )(C)


def _mm0_body(x_ref, w_ref, xw_ref):
    xw_ref[...] = jnp.dot(x_ref[...], w_ref[...],
                          preferred_element_type=jnp.float32)


def _mm0(x, W0):
    return pl.pallas_call(
        _mm0_body,
        grid=(NI,),
        in_specs=[pl.BlockSpec((BM, IN), lambda i: (i, 0)),
                  pl.BlockSpec((IN, HID), lambda i: (0, 0))],
        out_specs=pl.BlockSpec((BM, HID), lambda i: (i, 0)),
        out_shape=jax.ShapeDtypeStruct((N, HID), jnp.float32),
    )(x, W0)


def _conv1_body(c_ref, xwk_ref, dinvk_ref, dinv_ref, xw_ref, b_ref, pw_ref,
                x0_ref, srow_ref):
    k = pl.program_id(1)

    @pl.when(k == 0)
    def _():
        x0_ref[...] = jnp.zeros_like(x0_ref)

    x0_ref[...] += jnp.dot(c_ref[...].astype(jnp.float32),
                           dinvk_ref[...] * xwk_ref[...],
                           preferred_element_type=jnp.float32)

    @pl.when(k == NK - 1)
    def _():
        dinv = dinv_ref[...]
        xw = xw_ref[...]
        pre = dinv * x0_ref[...] + (2.0 * dinv * dinv) * xw + b_ref[...]
        x0 = jnp.maximum(pre, 0.0)
        x0_ref[...] = x0
        pw = pw_ref[...]
        pwn = pw * lax.rsqrt(jnp.sum(pw * pw))
        s_col = jnp.tanh(jnp.dot(x0, pwn, preferred_element_type=jnp.float32))
        srow_ref[...] = (s_col * _eye(BM)).sum(axis=0, keepdims=True)


def _conv1(C, dinv, xw0, b0r, pw2):
    return pl.pallas_call(
        _conv1_body,
        grid=(NI, NK),
        in_specs=[pl.BlockSpec((BM, BK), lambda i, k: (i, k)),
                  pl.BlockSpec((BK, HID), lambda i, k: (k, 0)),
                  pl.BlockSpec((BK, 1), lambda i, k: (k, 0)),
                  pl.BlockSpec((BM, 1), lambda i, k: (i, 0)),
                  pl.BlockSpec((BM, HID), lambda i, k: (i, 0)),
                  pl.BlockSpec((1, HID), lambda i, k: (0, 0)),
                  pl.BlockSpec((HID, 1), lambda i, k: (0, 0))],
        out_specs=(pl.BlockSpec((BM, HID), lambda i, k: (i, 0)),
                   pl.BlockSpec((1, BM), lambda i, k: (0, i))),
        out_shape=(jax.ShapeDtypeStruct((N, HID), jnp.float32),
                   jax.ShapeDtypeStruct((1, N), jnp.float32)),
    )(C, xw0, dinv, dinv, xw0, b0r, pw2)


def _pool_body(srow_ref, sblk_ref, x0_ref, w1_ref, mcol_ref, mrow_ref, y_ref):
    i = pl.program_id(0)
    s_all = srow_ref[...]
    s_blk = sblk_ref[...]
    s_col = (s_blk * _eye(BM)).sum(axis=1, keepdims=True)
    gt = (s_all > s_col).astype(jnp.float32).sum(axis=1, keepdims=True)
    jg = lax.broadcasted_iota(jnp.int32, (BM, N), 1)
    ig = lax.broadcasted_iota(jnp.int32, (BM, N), 0) + i * BM
    eqc = ((s_all == s_col) & (jg < ig)).astype(jnp.float32).sum(
        axis=1, keepdims=True)
    rank = gt + eqc
    m = (rank < float(K)).astype(jnp.float32)
    mcol_ref[...] = m
    mrow_ref[...] = (m * _eye(BM)).sum(axis=0, keepdims=True)
    y_ref[...] = jnp.dot(x0_ref[...] * (s_col * m), w1_ref[...],
                         preferred_element_type=jnp.float32)


def _pool(score_row, x0, W1):
    return pl.pallas_call(
        _pool_body,
        grid=(NI,),
        in_specs=[pl.BlockSpec((1, N), lambda i: (0, 0)),
                  pl.BlockSpec((1, BM), lambda i: (0, i)),
                  pl.BlockSpec((BM, HID), lambda i: (i, 0)),
                  pl.BlockSpec((HID, HID), lambda i: (0, 0))],
        out_specs=(pl.BlockSpec((BM, 1), lambda i: (i, 0)),
                   pl.BlockSpec((1, BM), lambda i: (0, i)),
                   pl.BlockSpec((BM, HID), lambda i: (i, 0))),
        out_shape=(jax.ShapeDtypeStruct((N, 1), jnp.float32),
                   jax.ShapeDtypeStruct((1, N), jnp.float32),
                   jax.ShapeDtypeStruct((N, HID), jnp.float32)),
    )(score_row, score_row, x0, W1)


def _stats_body(cik_ref, cki_ref, mrow_ref, mcol_ref, diag_ref,
                u1c_ref, u1r_ref, dfull_ref):
    i, k = pl.program_id(0), pl.program_id(1)

    @pl.when(k == 0)
    def _():
        u1c_ref[...] = jnp.zeros_like(u1c_ref)
        dfull_ref[...] = jnp.zeros_like(dfull_ref)

    cik = cik_ref[...].astype(jnp.float32)
    u1c_ref[...] += (cik * mrow_ref[...]).sum(axis=1, keepdims=True)
    prod = jnp.dot(cik_ref[...], cki_ref[...],
                   preferred_element_type=jnp.float32)
    dfull_ref[...] += (prod * _eye(BM)).sum(axis=1, keepdims=True)

    @pl.when(k == NK - 1)
    def _():
        c = diag_ref[...]
        u1 = u1c_ref[...] + (1.0 - c) * mcol_ref[...]
        u1c_ref[...] = u1
        u1r_ref[...] = (u1 * _eye(BM)).sum(axis=0, keepdims=True)
        dfull_ref[...] = 1.0 + dfull_ref[...] - c * c


def _stats(C, m_row, m_col, cdiag):
    return pl.pallas_call(
        _stats_body,
        grid=(NI, NK),
        in_specs=[pl.BlockSpec((BM, BK), lambda i, k: (i, k)),
                  pl.BlockSpec((BK, BM), lambda i, k: (k, i)),
                  pl.BlockSpec((1, BK), lambda i, k: (0, k)),
                  pl.BlockSpec((BM, 1), lambda i, k: (i, 0)),
                  pl.BlockSpec((BM, 1), lambda i, k: (i, 0))],
        out_specs=(pl.BlockSpec((BM, 1), lambda i, k: (i, 0)),
                   pl.BlockSpec((1, BM), lambda i, k: (0, i)),
                   pl.BlockSpec((BM, 1), lambda i, k: (i, 0))),
        out_shape=(jax.ShapeDtypeStruct((N, 1), jnp.float32),
                   jax.ShapeDtypeStruct((1, N), jnp.float32),
                   jax.ShapeDtypeStruct((N, 1), jnp.float32)),
    )(C, C, m_row, m_col, cdiag)


def _u2_body(c_ref, u1r_ref, u1c_ref, diag_ref, mcol_ref, dfull_ref, y_ref,
             dinvp_ref, z_ref):
    k = pl.program_id(1)

    @pl.when(k == 0)
    def _():
        dinvp_ref[...] = jnp.zeros_like(dinvp_ref)

    dinvp_ref[...] += (c_ref[...].astype(jnp.float32)
                       * u1r_ref[...]).sum(axis=1, keepdims=True)

    @pl.when(k == NK - 1)
    def _():
        c = diag_ref[...]
        u2 = dinvp_ref[...] + (1.0 - c) * u1c_ref[...]
        deg_pool = mcol_ref[...] * (u2 - dfull_ref[...]) + 2.0
        dinvp = lax.rsqrt(deg_pool)
        dinvp_ref[...] = dinvp
        z_ref[...] = dinvp * y_ref[...]


def _u2(C, u1r, u1c, cdiag, m_col, dfull, y):
    return pl.pallas_call(
        _u2_body,
        grid=(NI, NK),
        in_specs=[pl.BlockSpec((BM, BK), lambda i, k: (i, k)),
                  pl.BlockSpec((1, BK), lambda i, k: (0, k)),
                  pl.BlockSpec((BM, 1), lambda i, k: (i, 0)),
                  pl.BlockSpec((BM, 1), lambda i, k: (i, 0)),
                  pl.BlockSpec((BM, 1), lambda i, k: (i, 0)),
                  pl.BlockSpec((BM, 1), lambda i, k: (i, 0)),
                  pl.BlockSpec((BM, HID), lambda i, k: (i, 0))],
        out_specs=(pl.BlockSpec((BM, 1), lambda i, k: (i, 0)),
                   pl.BlockSpec((BM, HID), lambda i, k: (i, 0))),
        out_shape=(jax.ShapeDtypeStruct((N, 1), jnp.float32),
                   jax.ShapeDtypeStruct((N, HID), jnp.float32)),
    )(C, u1r, u1c, cdiag, m_col, dfull, y)


def _t1_body(c_ref, zk_ref, zi_ref, diag_ref, t1_ref):
    k = pl.program_id(1)

    @pl.when(k == 0)
    def _():
        t1_ref[...] = jnp.zeros_like(t1_ref)

    t1_ref[...] += jnp.dot(c_ref[...].astype(jnp.float32), zk_ref[...],
                           preferred_element_type=jnp.float32)

    @pl.when(k == NK - 1)
    def _():
        t1_ref[...] += (1.0 - diag_ref[...]) * zi_ref[...]


def _t1(C, z, cdiag):
    return pl.pallas_call(
        _t1_body,
        grid=(NI, NK),
        in_specs=[pl.BlockSpec((BM, BK), lambda i, k: (i, k)),
                  pl.BlockSpec((BK, HID), lambda i, k: (k, 0)),
                  pl.BlockSpec((BM, HID), lambda i, k: (i, 0)),
                  pl.BlockSpec((BM, 1), lambda i, k: (i, 0))],
        out_specs=pl.BlockSpec((BM, HID), lambda i, k: (i, 0)),
        out_shape=jax.ShapeDtypeStruct((N, HID), jnp.float32),
    )(C, z, z, cdiag)


def _t2_body(c_ref, t1k_ref, t1i_ref, diag_ref, dinvp_ref, dfull_ref,
             zi_ref, mcol_ref, x0_ref, b1_ref, xr_ref):
    k = pl.program_id(1)

    @pl.when(k == 0)
    def _():
        xr_ref[...] = jnp.zeros_like(xr_ref)

    xr_ref[...] += jnp.dot(c_ref[...].astype(jnp.float32), t1k_ref[...],
                           preferred_element_type=jnp.float32)

    @pl.when(k == NK - 1)
    def _():
        z = zi_ref[...]
        t2 = xr_ref[...] + (1.0 - diag_ref[...]) * t1i_ref[...]
        outp = dinvp_ref[...] * (t2 - dfull_ref[...] * z + 2.0 * z) + b1_ref[...]
        xr_ref[...] = x0_ref[...] + mcol_ref[...] * jnp.maximum(outp, 0.0)


def _t2(C, t1, cdiag, dinvp, dfull, z, m_col, x0, b1r):
    return pl.pallas_call(
        _t2_body,
        grid=(NI, NK),
        in_specs=[pl.BlockSpec((BM, BK), lambda i, k: (i, k)),
                  pl.BlockSpec((BK, HID), lambda i, k: (k, 0)),
                  pl.BlockSpec((BM, HID), lambda i, k: (i, 0)),
                  pl.BlockSpec((BM, 1), lambda i, k: (i, 0)),
                  pl.BlockSpec((BM, 1), lambda i, k: (i, 0)),
                  pl.BlockSpec((BM, 1), lambda i, k: (i, 0)),
                  pl.BlockSpec((BM, HID), lambda i, k: (i, 0)),
                  pl.BlockSpec((BM, 1), lambda i, k: (i, 0)),
                  pl.BlockSpec((BM, HID), lambda i, k: (i, 0)),
                  pl.BlockSpec((1, HID), lambda i, k: (0, 0))],
        out_specs=pl.BlockSpec((BM, HID), lambda i, k: (i, 0)),
        out_shape=jax.ShapeDtypeStruct((N, HID), jnp.float32),
    )(C, t1, t1, cdiag, dinvp, dfull, z, m_col, x0, b1r)


def _mm2_body(xr_ref, w2_ref, dinv_ref, xw2_ref, g_ref):
    xw2 = jnp.dot(xr_ref[...], w2_ref[...], preferred_element_type=jnp.float32)
    xw2_ref[...] = xw2
    g_ref[...] = dinv_ref[...] * xw2


def _mm2(xr, W2, dinv):
    return pl.pallas_call(
        _mm2_body,
        grid=(NI,),
        in_specs=[pl.BlockSpec((BM, HID), lambda i: (i, 0)),
                  pl.BlockSpec((HID, OUT), lambda i: (0, 0)),
                  pl.BlockSpec((BM, 1), lambda i: (i, 0))],
        out_specs=(pl.BlockSpec((BM, OUT), lambda i: (i, 0)),
                   pl.BlockSpec((BM, OUT), lambda i: (i, 0))),
        out_shape=(jax.ShapeDtypeStruct((N, OUT), jnp.float32),
                   jax.ShapeDtypeStruct((N, OUT), jnp.float32)),
    )(xr, W2, dinv)


def _conv3_body(c_ref, g_ref, dinv_ref, xw2_ref, b_ref, out_ref):
    k = pl.program_id(1)

    @pl.when(k == 0)
    def _():
        out_ref[...] = jnp.zeros_like(out_ref)

    out_ref[...] += jnp.dot(c_ref[...].astype(jnp.float32), g_ref[...],
                            preferred_element_type=jnp.float32)

    @pl.when(k == NK - 1)
    def _():
        dinv = dinv_ref[...]
        o = (dinv * out_ref[...] + (2.0 * dinv * dinv) * xw2_ref[...]
             + b_ref[...])
        mx = jnp.max(o, axis=1, keepdims=True)
        out_ref[...] = o - mx - jnp.log(
            jnp.sum(jnp.exp(o - mx), axis=1, keepdims=True))


def _conv3(C, g, dinv, xw2, b2r):
    return pl.pallas_call(
        _conv3_body,
        grid=(NI, NK),
        in_specs=[pl.BlockSpec((BM, BK), lambda i, k: (i, k)),
                  pl.BlockSpec((BK, OUT), lambda i, k: (k, 0)),
                  pl.BlockSpec((BM, 1), lambda i, k: (i, 0)),
                  pl.BlockSpec((BM, OUT), lambda i, k: (i, 0)),
                  pl.BlockSpec((1, OUT), lambda i, k: (0, 0))],
        out_specs=pl.BlockSpec((BM, OUT), lambda i, k: (i, 0)),
        out_shape=jax.ShapeDtypeStruct((N, OUT), jnp.float32),
    )(C, g, dinv, xw2, b2r)


def kernel(x, edge_index, W0, b0, pw, W1, b1, W2, b2):
    src = edge_index[0]
    dst = edge_index[1]
    cflat = _build_c(src, dst)
    C = cflat.reshape(N, N)

    xw0 = _mm0(x, W0)
    dinv, cdiag, Cbf = _prep(C)
    x0, score_row = _conv1(Cbf, dinv, xw0, b0.reshape(1, HID),
                           pw.reshape(HID, 1))
    m_col, m_row, y = _pool(score_row, x0, W1)
    u1c, u1r, dfull = _stats(Cbf, m_row, m_col, cdiag)
    dinvp, z = _u2(Cbf, u1r, u1c, cdiag, m_col, dfull, y)
    t1 = _t1(Cbf, z, cdiag)
    xr = _t2(Cbf, t1, cdiag, dinvp, dfull, z, m_col, x0, b1.reshape(1, HID))
    xw2, g = _mm2(xr, W2, dinv)
    return _conv3(Cbf, g, dinv, xw2, b2.reshape(1, OUT))
